# baseline (speedup 1.0000x reference)
import functools

import jax
import jax.numpy as jnp
from jax import lax
from jax.experimental import pallas as pl
from jax.experimental.pallas import tpu as pltpu
from jax.experimental.pallas import tpu_sc as plsc

VOCAB = 1000000
DIM = 64
BATCH = 16384

_INFO = plsc.get_sparse_core_info()
_NC = _INFO.num_cores
_NS = _INFO.num_subcores
_NW = _NC * _NS
_BPW = BATCH // _NW
_CHUNK = 128
_NCHUNK = _BPW // _CHUNK

_mesh = plsc.VectorSubcoreMesh(core_axis_name="c", subcore_axis_name="s")


@functools.partial(
    pl.kernel,
    mesh=_mesh,
    compiler_params=pltpu.CompilerParams(use_tc_tiling_on_sc=False),
    out_type=(
        jax.ShapeDtypeStruct((BATCH, DIM), jnp.float32),
        jax.ShapeDtypeStruct((BATCH, DIM), jnp.float32),
        jax.ShapeDtypeStruct((BATCH, 1), jnp.float32),
        jax.ShapeDtypeStruct((BATCH, 1), jnp.float32),
    ),
    scratch_types=[
        pltpu.VMEM((_NCHUNK, _CHUNK), jnp.int32),
        pltpu.VMEM((_NCHUNK, _CHUNK), jnp.int32),
        pltpu.VMEM((_BPW, DIM), jnp.float32),
        pltpu.VMEM((_BPW, DIM), jnp.float32),
        pltpu.VMEM((_BPW, 1), jnp.float32),
        pltpu.VMEM((_BPW, 1), jnp.float32),
        pltpu.SemaphoreType.DMA,
    ],
)
def _gather_sc(
    ii_hbm, jj_hbm, wi_hbm, wj_hbm, bi_hbm, bj_hbm,
    ri_hbm, rj_hbm, obi_hbm, obj_hbm,
    ii_v, jj_v, ri_v, rj_v, bi_v, bj_v, sem,
):
    wid = lax.axis_index("s") * _NC + lax.axis_index("c")
    base = wid * _BPW

    pltpu.sync_copy(ii_hbm.at[pl.ds(wid * _NCHUNK, _NCHUNK)], ii_v)
    pltpu.sync_copy(jj_hbm.at[pl.ds(wid * _NCHUNK, _NCHUNK)], jj_v)

    copies = []
    for k in range(_NCHUNK):
        rows = pl.ds(k * _CHUNK, _CHUNK)
        copies.append(pltpu.async_copy(wi_hbm.at[ii_v.at[k]], ri_v.at[rows], sem))
        copies.append(pltpu.async_copy(wj_hbm.at[jj_v.at[k]], rj_v.at[rows], sem))
        copies.append(pltpu.async_copy(bi_hbm.at[ii_v.at[k]], bi_v.at[rows], sem))
        copies.append(pltpu.async_copy(bj_hbm.at[jj_v.at[k]], bj_v.at[rows], sem))
    for c in copies:
        c.wait()

    pltpu.sync_copy(ri_v, ri_hbm.at[pl.ds(base, _BPW)])
    pltpu.sync_copy(rj_v, rj_hbm.at[pl.ds(base, _BPW)])
    pltpu.sync_copy(bi_v, obi_hbm.at[pl.ds(base, _BPW)])
    pltpu.sync_copy(bj_v, obj_hbm.at[pl.ds(base, _BPW)])


def _dot_tc(gi_ref, gj_ref, bi_ref, bj_ref, out_ref):
    prod = gi_ref[...] * gj_ref[...]
    out_ref[...] = jnp.sum(prod, axis=1) + bi_ref[...] + bj_ref[...]


_dot = pl.pallas_call(
    _dot_tc,
    grid=(BATCH // 512,),
    in_specs=[
        pl.BlockSpec((512, DIM), lambda i: (i, 0)),
        pl.BlockSpec((512, DIM), lambda i: (i, 0)),
        pl.BlockSpec((512,), lambda i: (i,)),
        pl.BlockSpec((512,), lambda i: (i,)),
    ],
    out_specs=pl.BlockSpec((512,), lambda i: (i,)),
    out_shape=jax.ShapeDtypeStruct((BATCH,), jnp.float32),
)


def kernel(i_indices, j_indices, wi, wj, bi, bj):
    ii = i_indices.astype(jnp.int32).reshape(_NW * _NCHUNK, _CHUNK)
    jj = j_indices.astype(jnp.int32).reshape(_NW * _NCHUNK, _CHUNK)
    gi, gj, bgi, bgj = _gather_sc(ii, jj, wi, wj, bi, bj)
    return _dot(gi, gj, bgi.reshape(BATCH), bgj.reshape(BATCH))


# speedup vs baseline: 2.5122x; 2.5122x over previous
import functools

import jax
import jax.numpy as jnp
from jax import lax
from jax.experimental import pallas as pl
from jax.experimental.pallas import tpu as pltpu
from jax.experimental.pallas import tpu_sc as plsc

VOCAB = 1000000
DIM = 64
BATCH = 16384

_INFO = plsc.get_sparse_core_info()
_NC = _INFO.num_cores
_NS = _INFO.num_subcores
_NW = _NC * _NS
_BPW = BATCH // _NW
_CHUNK = 128
_NCHUNK = _BPW // _CHUNK

_mesh = plsc.VectorSubcoreMesh(core_axis_name="c", subcore_axis_name="s")


@functools.partial(
    pl.kernel,
    mesh=_mesh,
    compiler_params=pltpu.CompilerParams(use_tc_tiling_on_sc=False),
    out_type=(
        jax.ShapeDtypeStruct((BATCH, DIM), jnp.float32),
        jax.ShapeDtypeStruct((BATCH, DIM), jnp.float32),
        jax.ShapeDtypeStruct((BATCH,), jnp.float32),
        jax.ShapeDtypeStruct((BATCH,), jnp.float32),
    ),
    scratch_types=[
        pltpu.VMEM((_NCHUNK, _CHUNK), jnp.int32),
        pltpu.VMEM((_NCHUNK, _CHUNK), jnp.int32),
        pltpu.VMEM((_BPW, DIM), jnp.float32),
        pltpu.VMEM((_BPW, DIM), jnp.float32),
        pltpu.VMEM((_BPW,), jnp.float32),
        pltpu.VMEM((_BPW,), jnp.float32),
        pltpu.SemaphoreType.DMA,
    ],
)
def _gather_sc(
    ii_hbm, jj_hbm, wi_hbm, wj_hbm, bi_hbm, bj_hbm,
    ri_hbm, rj_hbm, obi_hbm, obj_hbm,
    ii_v, jj_v, ri_v, rj_v, bi_v, bj_v, sem,
):
    wid = lax.axis_index("s") * _NC + lax.axis_index("c")
    base = wid * _BPW

    pltpu.sync_copy(ii_hbm.at[pl.ds(wid * _NCHUNK, _NCHUNK)], ii_v)
    pltpu.sync_copy(jj_hbm.at[pl.ds(wid * _NCHUNK, _NCHUNK)], jj_v)

    copies = []
    for k in range(_NCHUNK):
        rows = pl.ds(k * _CHUNK, _CHUNK)
        copies.append(pltpu.async_copy(wi_hbm.at[ii_v.at[k]], ri_v.at[rows], sem))
        copies.append(pltpu.async_copy(wj_hbm.at[jj_v.at[k]], rj_v.at[rows], sem))
        copies.append(pltpu.async_copy(bi_hbm.at[ii_v.at[k]], bi_v.at[rows], sem))
        copies.append(pltpu.async_copy(bj_hbm.at[jj_v.at[k]], bj_v.at[rows], sem))
    for c in copies:
        c.wait()

    pltpu.sync_copy(ri_v, ri_hbm.at[pl.ds(base, _BPW)])
    pltpu.sync_copy(rj_v, rj_hbm.at[pl.ds(base, _BPW)])
    pltpu.sync_copy(bi_v, obi_hbm.at[pl.ds(base, _BPW)])
    pltpu.sync_copy(bj_v, obj_hbm.at[pl.ds(base, _BPW)])


def _dot_tc(gi_ref, gj_ref, bi_ref, bj_ref, out_ref):
    prod = gi_ref[...] * gj_ref[...]
    out_ref[...] = jnp.sum(prod, axis=1) + bi_ref[...] + bj_ref[...]


_dot = pl.pallas_call(
    _dot_tc,
    grid=(BATCH // 512,),
    in_specs=[
        pl.BlockSpec((512, DIM), lambda i: (i, 0)),
        pl.BlockSpec((512, DIM), lambda i: (i, 0)),
        pl.BlockSpec((512,), lambda i: (i,)),
        pl.BlockSpec((512,), lambda i: (i,)),
    ],
    out_specs=pl.BlockSpec((512,), lambda i: (i,)),
    out_shape=jax.ShapeDtypeStruct((BATCH,), jnp.float32),
)


def kernel(i_indices, j_indices, wi, wj, bi, bj):
    ii = i_indices.astype(jnp.int32).reshape(_NW * _NCHUNK, _CHUNK)
    jj = j_indices.astype(jnp.int32).reshape(_NW * _NCHUNK, _CHUNK)
    gi, gj, bgi, bgj = _gather_sc(ii, jj, wi, wj, bi.reshape(VOCAB), bj.reshape(VOCAB))
    return _dot(gi, gj, bgi, bgj)


# speedup vs baseline: 2.5137x; 1.0006x over previous
import functools

import jax
import jax.numpy as jnp
from jax import lax
from jax.experimental import pallas as pl
from jax.experimental.pallas import tpu as pltpu
from jax.experimental.pallas import tpu_sc as plsc

VOCAB = 1000000
DIM = 64
BATCH = 16384

_INFO = plsc.get_sparse_core_info()
_NC = _INFO.num_cores
_NS = _INFO.num_subcores
_NW = _NC * _NS
_BPW = BATCH // _NW
_CHUNK = 128
_NCHUNK = _BPW // _CHUNK

_mesh = plsc.VectorSubcoreMesh(core_axis_name="c", subcore_axis_name="s")


@functools.partial(
    pl.kernel,
    mesh=_mesh,
    compiler_params=pltpu.CompilerParams(use_tc_tiling_on_sc=False),
    out_type=(
        jax.ShapeDtypeStruct((BATCH, DIM), jnp.float32),
        jax.ShapeDtypeStruct((BATCH, DIM), jnp.float32),
        jax.ShapeDtypeStruct((BATCH,), jnp.float32),
        jax.ShapeDtypeStruct((BATCH,), jnp.float32),
    ),
    scratch_types=[
        pltpu.VMEM((_NCHUNK, _CHUNK), jnp.int32),
        pltpu.VMEM((_NCHUNK, _CHUNK), jnp.int32),
        pltpu.VMEM((_BPW, DIM), jnp.float32),
        pltpu.VMEM((_BPW, DIM), jnp.float32),
        pltpu.VMEM((_BPW,), jnp.float32),
        pltpu.VMEM((_BPW,), jnp.float32),
        pltpu.SemaphoreType.DMA,
    ],
)
def _gather_sc(
    ii_hbm, jj_hbm, wi_hbm, wj_hbm, bi_hbm, bj_hbm,
    ri_hbm, rj_hbm, obi_hbm, obj_hbm,
    ii_v, jj_v, ri_v, rj_v, bi_v, bj_v, sem,
):
    wid = lax.axis_index("s") * _NC + lax.axis_index("c")
    base = wid * _BPW

    pltpu.sync_copy(ii_hbm.at[pl.ds(wid * _NCHUNK, _NCHUNK)], ii_v)
    pltpu.sync_copy(jj_hbm.at[pl.ds(wid * _NCHUNK, _NCHUNK)], jj_v)

    copies = []
    for k in range(_NCHUNK):
        rows = pl.ds(k * _CHUNK, _CHUNK)
        copies.append(pltpu.async_copy(wi_hbm.at[ii_v.at[k]], ri_v.at[rows], sem))
        copies.append(pltpu.async_copy(wj_hbm.at[jj_v.at[k]], rj_v.at[rows], sem))
        copies.append(pltpu.async_copy(bi_hbm.at[ii_v.at[k]], bi_v.at[rows], sem))
        copies.append(pltpu.async_copy(bj_hbm.at[jj_v.at[k]], bj_v.at[rows], sem))
    for c in copies:
        c.wait()

    pltpu.sync_copy(ri_v, ri_hbm.at[pl.ds(base, _BPW)])
    pltpu.sync_copy(rj_v, rj_hbm.at[pl.ds(base, _BPW)])
    pltpu.sync_copy(bi_v, obi_hbm.at[pl.ds(base, _BPW)])
    pltpu.sync_copy(bj_v, obj_hbm.at[pl.ds(base, _BPW)])


def _dot_tc(gi_ref, gj_ref, bi_ref, bj_ref, out_ref):
    prod = gi_ref[...] * gj_ref[...]
    out_ref[...] = jnp.sum(prod, axis=1) + bi_ref[...] + bj_ref[...]


_dot = pl.pallas_call(
    _dot_tc,
    grid=(BATCH // 512,),
    in_specs=[
        pl.BlockSpec((512, DIM), lambda i: (i, 0)),
        pl.BlockSpec((512, DIM), lambda i: (i, 0)),
        pl.BlockSpec((512,), lambda i: (i,)),
        pl.BlockSpec((512,), lambda i: (i,)),
    ],
    out_specs=pl.BlockSpec((512,), lambda i: (i,)),
    out_shape=jax.ShapeDtypeStruct((BATCH,), jnp.float32),
)


def kernel(i_indices, j_indices, wi, wj, bi, bj):
    ii = i_indices.astype(jnp.int32).reshape(_NW * _NCHUNK, _CHUNK)
    jj = j_indices.astype(jnp.int32).reshape(_NW * _NCHUNK, _CHUNK)
    bi_f = bi.reshape(-1)
    bj_f = bj.reshape(-1)
    gi, gj, bgi, bgj = _gather_sc(ii, jj, wi, wj, bi_f, bj_f)
    return _dot(gi, gj, bgi, bgj)
